# baseline (speedup 1.0000x reference)
import functools

import jax
import jax.numpy as jnp
from jax import lax
from jax.experimental import pallas as pl
from jax.experimental.pallas import tpu as pltpu
from jax.experimental.pallas import tpu_sc as plsc

H = 256
HALF = 128
T = 20
NPG = 1000
EPG = 5000
N = T * NPG
E = T * EPG
NUM_REL = 200

_NC = 2
_NS = 16
_CE = 128
_ZR = 128

_GPR = 2
_RN = _GPR * NPG
_RE = _GPR * EPG
_ROUNDS = T // (_NC * _GPR)
_FULL = _RE // _CE
_KPT = (_FULL + _NS - 1) // _NS


def _part_copy(s, total, fn):
    nfull, tail = divmod(total, 128)
    for k in range((nfull + _NS - 1) // _NS):
        chunk = k * _NS + s

        @pl.when(chunk < nfull)
        def _():
            fn(chunk * 128, 128)
    if tail:

        @pl.when(s == _NS - 1)
        def _():
            fn(nfull * 128, tail)


def _sc_mesh():
    return plsc.VectorSubcoreMesh(core_axis_name="c", subcore_axis_name="s")


_SC_PARAMS = pltpu.CompilerParams(use_tc_tiling_on_sc=False)


_GC = 128


def _gather_body(ids_hbm, tab_hbm, out_hbm, idxv, rows, idx32, rows32, sem):
    c = lax.axis_index("c")
    s = lax.axis_index("s")
    w = s * _NC + c
    n_full = N // _GC
    for k in range(5):
        chunk = k * 32 + w

        @pl.when(chunk < n_full)
        def _():
            off = pl.multiple_of(chunk * _GC, 8)
            pltpu.sync_copy(ids_hbm.at[pl.ds(off, _GC)], idxv)
            pltpu.async_copy(tab_hbm.at[idxv], rows, sem).wait()
            pltpu.sync_copy(rows, out_hbm.at[pl.ds(off, _GC)])

    @pl.when(w == 30)
    def _():
        base = n_full * _GC
        pltpu.sync_copy(ids_hbm.at[pl.ds(base, 32)], idx32)
        pltpu.async_copy(tab_hbm.at[idx32], rows32, sem).wait()
        pltpu.sync_copy(rows32, out_hbm.at[pl.ds(base, 32)])


def _gather_h0(node_ids, ent_embeds):
    return pl.kernel(
        _gather_body,
        out_type=jax.ShapeDtypeStruct((N, H), jnp.float32),
        mesh=_sc_mesh(),
        compiler_params=_SC_PARAMS,
        scratch_types=[
            pltpu.VMEM((_GC,), jnp.int32),
            pltpu.VMEM((_GC, H), jnp.float32),
            pltpu.VMEM((32,), jnp.int32),
            pltpu.VMEM((32, H), jnp.float32),
            pltpu.SemaphoreType.DMA,
        ],
    )(node_ids, ent_embeds)


def _edge_body(hl, aux, *refs):
    if aux:
        (src_hbm, dstl_hbm, typ_hbm, pidx_hbm, ids_hbm, tab_hbm,
         negrel_hbm, z_hbm, z16_hbm, o16_hbm,
         h_out, s_out, deg_out, p_out,
         acc, degsp, psp, relsp, onesb,
         srcv, dstl, typv, pidxv,
         srcv6, dstl6, typv6, pidxv6,
         hrows, rrows, hrows6, rrows6, gsem0, gsem1) = refs
        h_hbm = h_out
    else:
        (src_hbm, dstl_hbm, typ_hbm, h_hbm, negrel_hbm, z_hbm,
         s_out,
         acc, relsp,
         srcv, dstl, typv,
         srcv6, dstl6, typv6,
         hrows, rrows, hrows6, rrows6, gsem0, gsem1) = refs
        pidx_hbm = pidxv = pidxv6 = None

    c = lax.axis_index("c")
    s = lax.axis_index("s")

    @pl.when(s == 0)
    def _():
        pltpu.sync_copy(negrel_hbm, relsp)

    if aux:
        pltpu.sync_copy(o16_hbm, onesb)
        _part_copy(s, T * NUM_REL + 8,
                   lambda off, n: pltpu.sync_copy(z16_hbm.at[pl.ds(0, n)],
                                                  psp.at[pl.ds(off, n)]))
        half = N // _NC
        for k in range(5):
            chunk = k * _NS + s

            @pl.when(chunk < half // _GC)
            def _():
                off = pl.multiple_of(c * half + chunk * _GC, 8)
                pltpu.sync_copy(ids_hbm.at[pl.ds(off, _GC)], srcv)
                pltpu.async_copy(tab_hbm.at[srcv], hrows, gsem0).wait()
                pltpu.sync_copy(hrows, h_out.at[pl.ds(off, _GC)])

        @pl.when(s == _NS - 1)
        def _():
            off = pl.multiple_of(c * half + (half // _GC) * _GC, 8)
            pltpu.sync_copy(ids_hbm.at[pl.ds(off, 16)], srcv6)
            pltpu.async_copy(tab_hbm.at[srcv6], hrows6, gsem0).wait()
            pltpu.sync_copy(hrows6, h_out.at[pl.ds(off, 16)])

    def proc(n, off, sv, dl, tv, pv, hr, rr):
        pltpu.sync_copy(src_hbm.at[pl.ds(off, n)], sv)
        pltpu.sync_copy(dstl_hbm.at[pl.ds(off, n)], dl)
        pltpu.sync_copy(typ_hbm.at[pl.ds(off, n)], tv)
        if aux:
            pltpu.sync_copy(pidx_hbm.at[pl.ds(off, n)], pv)
        gh = pltpu.async_copy(h_hbm.at[sv], hr, gsem0)
        gr = pltpu.async_copy(relsp.at[tv], rr, gsem1)
        if aux:
            pltpu.sync_copy(onesb.at[pl.ds(0, n)], degsp.at[dl], add=True)
            pltpu.sync_copy(onesb.at[pl.ds(0, n)], psp.at[pv], add=True)
        gh.wait()
        gr.wait()
        pltpu.sync_copy(hr, acc.at[dl], add=True)
        pltpu.sync_copy(rr, acc.at[dl], add=True)

    for r in range(_ROUNDS):
        node_base = c * (N // _NC) + r * _RN
        edge_base = c * (E // _NC) + r * _RE

        _part_copy(s, _RN,
                   lambda off, n: pltpu.sync_copy(z_hbm.at[pl.ds(0, n)],
                                                  acc.at[pl.ds(off, n)]))
        if aux:
            _part_copy(s, _RN,
                       lambda off, n: pltpu.sync_copy(z16_hbm.at[pl.ds(0, n)],
                                                      degsp.at[pl.ds(off, n)]))
        plsc.subcore_barrier()

        for k in range(_KPT):
            chunk = k * _NS + s

            @pl.when(chunk < _FULL)
            def _():
                off = pl.multiple_of(edge_base + chunk * _CE, 8)
                proc(_CE, off, srcv, dstl, typv,
                     pidxv, hrows, rrows)

        @pl.when(s == _NS - 1)
        def _():
            off = pl.multiple_of(edge_base + _FULL * _CE, 8)
            proc(16, off, srcv6, dstl6, typv6,
                 pidxv6, hrows6, rrows6)

        plsc.subcore_barrier()

        _part_copy(s, _RN,
                   lambda off, n: pltpu.sync_copy(
                       acc.at[pl.ds(off, n)],
                       s_out.at[pl.ds(node_base + off, n)]))
        if aux:
            _part_copy(s, _RN,
                       lambda off, n: pltpu.sync_copy(
                           degsp.at[pl.ds(off, n)],
                           deg_out.at[pl.ds(node_base + off, n)]))
        plsc.subcore_barrier()

    if aux:
        pbase = c * (T * NUM_REL // _NC)
        _part_copy(s, T * NUM_REL // _NC,
                   lambda off, n: pltpu.sync_copy(
                       psp.at[pl.ds(pbase + off, n)],
                       p_out.at[pl.ds(pbase + off, n)]))


def _edge_pass(h, negrel, src, dstl, typ, pidx, aux, ids=None, tab=None):
    hl = H if aux else h.shape[1]
    zeros = jnp.zeros((_ZR, hl), jnp.float32)
    idxbufs = [pltpu.VMEM((_CE,), jnp.int32) for _ in range(4 if aux else 3)]
    idx6bufs = [pltpu.VMEM((16,), jnp.int32) for _ in range(4 if aux else 3)]
    rowbufs = [
        pltpu.VMEM((_CE, hl), jnp.float32),
        pltpu.VMEM((_CE, hl), jnp.float32),
        pltpu.VMEM((16, hl), jnp.float32),
        pltpu.VMEM((16, hl), jnp.float32),
        pltpu.SemaphoreType.DMA,
        pltpu.SemaphoreType.DMA,
    ]
    if aux:
        zeros16 = jnp.zeros((_ZR, 16), jnp.float32)
        ones16 = jnp.ones((_CE, 16), jnp.float32)
        return pl.kernel(
            functools.partial(_edge_body, hl, True),
            out_type=[
                jax.ShapeDtypeStruct((N, H), jnp.float32),
                jax.ShapeDtypeStruct((N, hl), jnp.float32),
                jax.ShapeDtypeStruct((N, 16), jnp.float32),
                jax.ShapeDtypeStruct((T * NUM_REL, 16), jnp.float32),
            ],
            mesh=_sc_mesh(),
            compiler_params=_SC_PARAMS,
            scratch_types=(
                [
                    pltpu.VMEM_SHARED((_RN, hl), jnp.float32),
                    pltpu.VMEM_SHARED((_RN, 16), jnp.float32),
                    pltpu.VMEM_SHARED((T * NUM_REL + 8, 16), jnp.float32),
                    pltpu.VMEM_SHARED((NUM_REL, hl), jnp.float32),
                    pltpu.VMEM((_CE, 16), jnp.float32),
                ]
                + idxbufs + idx6bufs + rowbufs
            ),
        )(src, dstl, typ, pidx, ids, tab, negrel, zeros, zeros16, ones16)
    return pl.kernel(
        functools.partial(_edge_body, hl, False),
        out_type=jax.ShapeDtypeStruct((N, hl), jnp.float32),
        mesh=_sc_mesh(),
        compiler_params=_SC_PARAMS,
        scratch_types=(
            [
                pltpu.VMEM_SHARED((_RN, hl), jnp.float32),
                pltpu.VMEM_SHARED((NUM_REL, hl), jnp.float32),
            ]
            + idxbufs + idx6bufs + rowbufs
        ),
    )(src, dstl, typ, h, negrel, zeros)


def _rel_prep_body(rel_ref, w1r_ref, w2r_ref, nr0_ref, er1_ref, nr1_ref, er2_ref):
    rel = rel_ref[...]
    nr0_ref[...] = -rel
    er1 = jax.nn.relu(jnp.dot(rel, w1r_ref[...], preferred_element_type=jnp.float32))
    er1_ref[...] = er1
    nr1_ref[...] = -er1
    er2_ref[...] = jax.nn.relu(jnp.dot(er1, w2r_ref[...], preferred_element_type=jnp.float32))


def _rel_prep(rel_embeds, W1_rel, W2_rel):
    return pl.pallas_call(
        _rel_prep_body,
        out_shape=[
            jax.ShapeDtypeStruct((NUM_REL, H), jnp.float32),
            jax.ShapeDtypeStruct((NUM_REL, HALF), jnp.float32),
            jax.ShapeDtypeStruct((NUM_REL, HALF), jnp.float32),
            jax.ShapeDtypeStruct((NUM_REL, H), jnp.float32),
        ],
    )(rel_embeds, W1_rel, W2_rel)


_BR = 400


def _layer_body(s_ref, h_ref, deg_ref, win_ref, wloop_ref, out_ref):
    inv = 1.0 / jnp.maximum(deg_ref[...][:, :1], 1.0)
    sc = s_ref[...] * inv
    out_ref[...] = jax.nn.relu(
        jnp.dot(sc, win_ref[...], preferred_element_type=jnp.float32)
        + jnp.dot(h_ref[...], wloop_ref[...], preferred_element_type=jnp.float32)
    )


def _fused_layer(S, h, deg16, W_in, W_loop):
    n, hin = S.shape
    hout = W_in.shape[1]
    return pl.pallas_call(
        _layer_body,
        grid=(n // _BR,),
        in_specs=[
            pl.BlockSpec((_BR, hin), lambda i: (i, 0)),
            pl.BlockSpec((_BR, hin), lambda i: (i, 0)),
            pl.BlockSpec((_BR, 16), lambda i: (i, 0)),
            pl.BlockSpec((hin, hout), lambda i: (0, 0)),
            pl.BlockSpec((hin, hout), lambda i: (0, 0)),
        ],
        out_specs=pl.BlockSpec((_BR, hout), lambda i: (i, 0)),
        out_shape=jax.ShapeDtypeStruct((n, hout), jnp.float32),
    )(S, h, deg16, W_in, W_loop)


def _pool_body(h_ref, p_ref, er2_ref, npool_ref, epool_ref):
    npool_ref[...] = jnp.max(h_ref[0], axis=0, keepdims=True)[None]
    mask = p_ref[0][:, :1] > 0.0
    masked = jnp.where(mask, er2_ref[...], -jnp.inf)
    ep = jnp.max(masked, axis=0, keepdims=True)
    epool_ref[...] = jnp.where(jnp.isfinite(ep), ep, 0.0)[None]


def _pools(h2, P, er2):
    npool, epool = pl.pallas_call(
        _pool_body,
        grid=(T,),
        in_specs=[
            pl.BlockSpec((1, NPG, H), lambda g: (g, 0, 0)),
            pl.BlockSpec((1, NUM_REL, 16), lambda g: (g, 0, 0)),
            pl.BlockSpec((NUM_REL, H), lambda g: (0, 0)),
        ],
        out_specs=[
            pl.BlockSpec((1, 1, H), lambda g: (g, 0, 0)),
            pl.BlockSpec((1, 1, H), lambda g: (g, 0, 0)),
        ],
        out_shape=[
            jax.ShapeDtypeStruct((T, 1, H), jnp.float32),
            jax.ShapeDtypeStruct((T, 1, H), jnp.float32),
        ],
    )(h2.reshape(T, NPG, H), P.reshape(T, NUM_REL, 16), er2)
    return npool.reshape(T, H), epool.reshape(T, H)


def kernel(node_ids, edge_index, edge_type, node_graph_id, edge_graph_id,
           time_idx, seq_mask, ent_embeds, rel_embeds,
           W1_in, W1_loop, W1_rel, W2_in, W2_loop, W2_rel):
    src = edge_index[0].astype(jnp.int32)
    dst = edge_index[1].astype(jnp.int32)
    typ = edge_type.astype(jnp.int32)
    dstl = dst % _RN
    pidx = (dst // NPG) * NUM_REL + typ
    negrel0, er1, negrel1, er2 = _rel_prep(rel_embeds, W1_rel, W2_rel)
    h0, S1, deg16, P = _edge_pass(None, negrel0, src, dstl, typ, pidx,
                                  aux=True, ids=node_ids.astype(jnp.int32),
                                  tab=ent_embeds)
    h1 = _fused_layer(S1, h0, deg16, W1_in, W1_loop)
    S2 = _edge_pass(h1, negrel1, src, dstl, typ, None, aux=False)
    h2 = _fused_layer(S2, h1, deg16, W2_in, W2_loop)
    node_pool, edge_pool = _pools(h2, P, er2)
    gi = jnp.concatenate([node_pool, edge_pool], axis=-1)
    embed_seq = gi[time_idx] * seq_mask[..., None]
    len_non_zero = jnp.sum(seq_mask, axis=-1).astype(jnp.int32)
    return embed_seq, len_non_zero


# speedup vs baseline: 1.0296x; 1.0296x over previous
import functools

import jax
import jax.numpy as jnp
from jax import lax
from jax.experimental import pallas as pl
from jax.experimental.pallas import tpu as pltpu
from jax.experimental.pallas import tpu_sc as plsc

H = 256
HALF = 128
T = 20
NPG = 1000
EPG = 5000
N = T * NPG
E = T * EPG
NUM_REL = 200

_NC = 2
_NS = 16
_CE = 128
_ZR = 128

_GPR = 2
_RN = _GPR * NPG
_RE = _GPR * EPG
_ROUNDS = T // (_NC * _GPR)
_FULL = _RE // _CE
_KPT = (_FULL + _NS - 1) // _NS


def _part_copy(s, total, fn):
    nfull, tail = divmod(total, 128)
    for k in range((nfull + _NS - 1) // _NS):
        chunk = k * _NS + s

        @pl.when(chunk < nfull)
        def _():
            fn(chunk * 128, 128)
    if tail:

        @pl.when(s == _NS - 1)
        def _():
            fn(nfull * 128, tail)


def _sc_mesh():
    return plsc.VectorSubcoreMesh(core_axis_name="c", subcore_axis_name="s")


_SC_PARAMS = pltpu.CompilerParams(use_tc_tiling_on_sc=False)


_GC = 128


def _gather_body(ids_hbm, tab_hbm, out_hbm, idxv, rows, idx32, rows32, sem):
    c = lax.axis_index("c")
    s = lax.axis_index("s")
    w = s * _NC + c
    n_full = N // _GC
    for k in range(5):
        chunk = k * 32 + w

        @pl.when(chunk < n_full)
        def _():
            off = pl.multiple_of(chunk * _GC, 8)
            pltpu.sync_copy(ids_hbm.at[pl.ds(off, _GC)], idxv)
            pltpu.async_copy(tab_hbm.at[idxv], rows, sem).wait()
            pltpu.sync_copy(rows, out_hbm.at[pl.ds(off, _GC)])

    @pl.when(w == 30)
    def _():
        base = n_full * _GC
        pltpu.sync_copy(ids_hbm.at[pl.ds(base, 32)], idx32)
        pltpu.async_copy(tab_hbm.at[idx32], rows32, sem).wait()
        pltpu.sync_copy(rows32, out_hbm.at[pl.ds(base, 32)])


def _gather_h0(node_ids, ent_embeds):
    return pl.kernel(
        _gather_body,
        out_type=jax.ShapeDtypeStruct((N, H), jnp.float32),
        mesh=_sc_mesh(),
        compiler_params=_SC_PARAMS,
        scratch_types=[
            pltpu.VMEM((_GC,), jnp.int32),
            pltpu.VMEM((_GC, H), jnp.float32),
            pltpu.VMEM((32,), jnp.int32),
            pltpu.VMEM((32, H), jnp.float32),
            pltpu.SemaphoreType.DMA,
        ],
    )(node_ids, ent_embeds)


def _edge_body(hl, aux, *refs):
    if aux:
        (src_hbm, dstl_hbm, typ_hbm, pidx_hbm, h_hbm, negrel_hbm, z_hbm,
         z16_hbm, o16_hbm,
         s_out, deg_out, p_out,
         acc, degsp, psp, relsp, onesb,
         srcv, dstl, typv, pidxv,
         srcv6, dstl6, typv6, pidxv6,
         hrows, rrows, hrows6, rrows6, gsem0, gsem1) = refs
    else:
        (src_hbm, dstl_hbm, typ_hbm, h_hbm, negrel_hbm, z_hbm,
         s_out,
         acc, relsp,
         srcv, dstl, typv,
         srcv6, dstl6, typv6,
         hrows, rrows, hrows6, rrows6, gsem0, gsem1) = refs
        pidx_hbm = pidxv = pidxv6 = None

    c = lax.axis_index("c")
    s = lax.axis_index("s")

    @pl.when(s == 0)
    def _():
        pltpu.sync_copy(negrel_hbm, relsp)

    if aux:
        pltpu.sync_copy(o16_hbm, onesb)
        _part_copy(s, T * NUM_REL + 8,
                   lambda off, n: pltpu.sync_copy(z16_hbm.at[pl.ds(0, n)],
                                                  psp.at[pl.ds(off, n)]))

    def proc(n, off, sv, dl, tv, pv, hr, rr):
        pltpu.sync_copy(src_hbm.at[pl.ds(off, n)], sv)
        pltpu.sync_copy(dstl_hbm.at[pl.ds(off, n)], dl)
        pltpu.sync_copy(typ_hbm.at[pl.ds(off, n)], tv)
        if aux:
            pltpu.sync_copy(pidx_hbm.at[pl.ds(off, n)], pv)
        gh = pltpu.async_copy(h_hbm.at[sv], hr, gsem0)
        gr = pltpu.async_copy(relsp.at[tv], rr, gsem1)
        if aux:
            pltpu.sync_copy(onesb.at[pl.ds(0, n)], degsp.at[dl], add=True)
            pltpu.sync_copy(onesb.at[pl.ds(0, n)], psp.at[pv], add=True)
        gh.wait()
        gr.wait()
        pltpu.sync_copy(hr, acc.at[dl], add=True)
        pltpu.sync_copy(rr, acc.at[dl], add=True)

    for r in range(_ROUNDS):
        node_base = c * (N // _NC) + r * _RN
        edge_base = c * (E // _NC) + r * _RE

        _part_copy(s, _RN,
                   lambda off, n: pltpu.sync_copy(z_hbm.at[pl.ds(0, n)],
                                                  acc.at[pl.ds(off, n)]))
        if aux:
            _part_copy(s, _RN,
                       lambda off, n: pltpu.sync_copy(z16_hbm.at[pl.ds(0, n)],
                                                      degsp.at[pl.ds(off, n)]))
        plsc.subcore_barrier()

        for k in range(_KPT):
            chunk = k * _NS + s

            @pl.when(chunk < _FULL)
            def _():
                off = pl.multiple_of(edge_base + chunk * _CE, 8)
                proc(_CE, off, srcv, dstl, typv,
                     pidxv, hrows, rrows)

        @pl.when(s == _NS - 1)
        def _():
            off = pl.multiple_of(edge_base + _FULL * _CE, 8)
            proc(16, off, srcv6, dstl6, typv6,
                 pidxv6, hrows6, rrows6)

        plsc.subcore_barrier()

        _part_copy(s, _RN,
                   lambda off, n: pltpu.sync_copy(
                       acc.at[pl.ds(off, n)],
                       s_out.at[pl.ds(node_base + off, n)]))
        if aux:
            _part_copy(s, _RN,
                       lambda off, n: pltpu.sync_copy(
                           degsp.at[pl.ds(off, n)],
                           deg_out.at[pl.ds(node_base + off, n)]))
        plsc.subcore_barrier()

    if aux:
        pbase = c * (T * NUM_REL // _NC)
        _part_copy(s, T * NUM_REL // _NC,
                   lambda off, n: pltpu.sync_copy(
                       psp.at[pl.ds(pbase + off, n)],
                       p_out.at[pl.ds(pbase + off, n)]))


def _edge_pass(h, negrel, src, dstl, typ, pidx, aux):
    hl = h.shape[1]
    zeros = jnp.zeros((_ZR, hl), jnp.float32)
    idxbufs = [pltpu.VMEM((_CE,), jnp.int32) for _ in range(4 if aux else 3)]
    idx6bufs = [pltpu.VMEM((16,), jnp.int32) for _ in range(4 if aux else 3)]
    rowbufs = [
        pltpu.VMEM((_CE, hl), jnp.float32),
        pltpu.VMEM((_CE, hl), jnp.float32),
        pltpu.VMEM((16, hl), jnp.float32),
        pltpu.VMEM((16, hl), jnp.float32),
        pltpu.SemaphoreType.DMA,
        pltpu.SemaphoreType.DMA,
    ]
    if aux:
        zeros16 = jnp.zeros((_ZR, 16), jnp.float32)
        ones16 = jnp.ones((_CE, 16), jnp.float32)
        return pl.kernel(
            functools.partial(_edge_body, hl, True),
            out_type=[
                jax.ShapeDtypeStruct((N, hl), jnp.float32),
                jax.ShapeDtypeStruct((N, 16), jnp.float32),
                jax.ShapeDtypeStruct((T * NUM_REL, 16), jnp.float32),
            ],
            mesh=_sc_mesh(),
            compiler_params=_SC_PARAMS,
            scratch_types=(
                [
                    pltpu.VMEM_SHARED((_RN, hl), jnp.float32),
                    pltpu.VMEM_SHARED((_RN, 16), jnp.float32),
                    pltpu.VMEM_SHARED((T * NUM_REL + 8, 16), jnp.float32),
                    pltpu.VMEM_SHARED((NUM_REL, hl), jnp.float32),
                    pltpu.VMEM((_CE, 16), jnp.float32),
                ]
                + idxbufs + idx6bufs + rowbufs
            ),
        )(src, dstl, typ, pidx, h, negrel, zeros, zeros16, ones16)
    return pl.kernel(
        functools.partial(_edge_body, hl, False),
        out_type=jax.ShapeDtypeStruct((N, hl), jnp.float32),
        mesh=_sc_mesh(),
        compiler_params=_SC_PARAMS,
        scratch_types=(
            [
                pltpu.VMEM_SHARED((_RN, hl), jnp.float32),
                pltpu.VMEM_SHARED((NUM_REL, hl), jnp.float32),
            ]
            + idxbufs + idx6bufs + rowbufs
        ),
    )(src, dstl, typ, h, negrel, zeros)


def _rel_prep_body(rel_ref, w1r_ref, w2r_ref, nr0_ref, er1_ref, nr1_ref, er2_ref):
    rel = rel_ref[...]
    nr0_ref[...] = -rel
    er1 = jax.nn.relu(jnp.dot(rel, w1r_ref[...], preferred_element_type=jnp.float32))
    er1_ref[...] = er1
    nr1_ref[...] = -er1
    er2_ref[...] = jax.nn.relu(jnp.dot(er1, w2r_ref[...], preferred_element_type=jnp.float32))


def _rel_prep(rel_embeds, W1_rel, W2_rel):
    return pl.pallas_call(
        _rel_prep_body,
        out_shape=[
            jax.ShapeDtypeStruct((NUM_REL, H), jnp.float32),
            jax.ShapeDtypeStruct((NUM_REL, HALF), jnp.float32),
            jax.ShapeDtypeStruct((NUM_REL, HALF), jnp.float32),
            jax.ShapeDtypeStruct((NUM_REL, H), jnp.float32),
        ],
    )(rel_embeds, W1_rel, W2_rel)


_BR = 400


def _layer_body(s_ref, h_ref, deg_ref, win_ref, wloop_ref, out_ref):
    inv = 1.0 / jnp.maximum(deg_ref[...][:, :1], 1.0)
    sc = s_ref[...] * inv
    out_ref[...] = jax.nn.relu(
        jnp.dot(sc, win_ref[...], preferred_element_type=jnp.float32)
        + jnp.dot(h_ref[...], wloop_ref[...], preferred_element_type=jnp.float32)
    )


def _fused_layer(S, h, deg16, W_in, W_loop):
    n, hin = S.shape
    hout = W_in.shape[1]
    return pl.pallas_call(
        _layer_body,
        grid=(n // _BR,),
        in_specs=[
            pl.BlockSpec((_BR, hin), lambda i: (i, 0)),
            pl.BlockSpec((_BR, hin), lambda i: (i, 0)),
            pl.BlockSpec((_BR, 16), lambda i: (i, 0)),
            pl.BlockSpec((hin, hout), lambda i: (0, 0)),
            pl.BlockSpec((hin, hout), lambda i: (0, 0)),
        ],
        out_specs=pl.BlockSpec((_BR, hout), lambda i: (i, 0)),
        out_shape=jax.ShapeDtypeStruct((n, hout), jnp.float32),
    )(S, h, deg16, W_in, W_loop)


def _pool_body(h_ref, p_ref, er2_ref, npool_ref, epool_ref):
    npool_ref[...] = jnp.max(h_ref[0], axis=0, keepdims=True)[None]
    mask = p_ref[0][:, :1] > 0.0
    masked = jnp.where(mask, er2_ref[...], -jnp.inf)
    ep = jnp.max(masked, axis=0, keepdims=True)
    epool_ref[...] = jnp.where(jnp.isfinite(ep), ep, 0.0)[None]


def _pools(h2, P, er2):
    npool, epool = pl.pallas_call(
        _pool_body,
        grid=(T,),
        in_specs=[
            pl.BlockSpec((1, NPG, H), lambda g: (g, 0, 0)),
            pl.BlockSpec((1, NUM_REL, 16), lambda g: (g, 0, 0)),
            pl.BlockSpec((NUM_REL, H), lambda g: (0, 0)),
        ],
        out_specs=[
            pl.BlockSpec((1, 1, H), lambda g: (g, 0, 0)),
            pl.BlockSpec((1, 1, H), lambda g: (g, 0, 0)),
        ],
        out_shape=[
            jax.ShapeDtypeStruct((T, 1, H), jnp.float32),
            jax.ShapeDtypeStruct((T, 1, H), jnp.float32),
        ],
    )(h2.reshape(T, NPG, H), P.reshape(T, NUM_REL, 16), er2)
    return npool.reshape(T, H), epool.reshape(T, H)


def kernel(node_ids, edge_index, edge_type, node_graph_id, edge_graph_id,
           time_idx, seq_mask, ent_embeds, rel_embeds,
           W1_in, W1_loop, W1_rel, W2_in, W2_loop, W2_rel):
    src = edge_index[0].astype(jnp.int32)
    dst = edge_index[1].astype(jnp.int32)
    typ = edge_type.astype(jnp.int32)
    dstl = dst % _RN
    pidx = (dst // NPG) * NUM_REL + typ
    negrel0, er1, negrel1, er2 = _rel_prep(rel_embeds, W1_rel, W2_rel)
    h0 = _gather_h0(node_ids.astype(jnp.int32), ent_embeds)
    S1, deg16, P = _edge_pass(h0, negrel0, src, dstl, typ, pidx, aux=True)
    h1 = _fused_layer(S1, h0, deg16, W1_in, W1_loop)
    S2 = _edge_pass(h1, negrel1, src, dstl, typ, None, aux=False)
    h2 = _fused_layer(S2, h1, deg16, W2_in, W2_loop)
    node_pool, edge_pool = _pools(h2, P, er2)
    gi = jnp.concatenate([node_pool, edge_pool], axis=-1)
    embed_seq = gi[time_idx] * seq_mask[..., None]
    len_non_zero = jnp.sum(seq_mask, axis=-1).astype(jnp.int32)
    return embed_seq, len_non_zero
